# baseline (speedup 1.0000x reference)
import functools

import jax
import jax.numpy as jnp
from jax import lax
from jax.experimental import pallas as pl
from jax.experimental.pallas import tpu as pltpu
from jax.experimental.pallas import tpu_sc as plsc

B, IN, N, C, TAU = 4096, 1024, 8192, 16, 10.0
NC, NS, LANES = 2, 16, 16
NW = NC * NS
NPW = N // NW
G = 4
NCHUNK = NPW // G


def _compute_coeffs(w_v, c0_v, ca_v, cb_v, cab_v):
    def group(q, carry):
        sl = pl.ds(q * LANES, LANES)
        rows = [w_v[g, sl] for g in range(16)]
        m = rows[0]
        for g in range(1, 16):
            m = jnp.maximum(m, rows[g])
        e = [jnp.exp(r - m) for r in rows]
        s = e[0]
        for g in range(1, 16):
            s = s + e[g]
        inv = 1.0 / s
        c0r = ((e[8] + e[9]) + (e[10] + e[11])) + ((e[12] + e[13]) + (e[14] + e[15]))
        car = ((e[2] + e[3]) + (e[6] + e[7])) - ((e[8] + e[9]) + (e[12] + e[13]))
        cbr = ((e[4] + e[5]) + (e[6] + e[7])) - ((e[8] + e[9]) + (e[10] + e[11]))
        cabr = (((e[1] + e[8]) + 2.0 * e[9]) + (e[11] + e[13])) - \
               (((e[2] + e[4]) + 2.0 * e[6]) + (e[7] + e[14]))
        c0_v[sl] = c0r * inv
        ca_v[sl] = car * inv
        cb_v[sl] = cbr * inv
        cab_v[sl] = cabr * inv
        return carry

    lax.fori_loop(0, NPW // LANES, group, 0)


def _make_layer(in_dim, final):
    mesh = plsc.VectorSubcoreMesh(core_axis_name="c", subcore_axis_name="s")
    if final:
        out_t = jax.ShapeDtypeStruct((2, C, B), jnp.float32)
        o_scratch = [pltpu.VMEM((B,), jnp.float32)]
    else:
        out_t = jax.ShapeDtypeStruct((N, B), jnp.float32)
        o_scratch = [pltpu.VMEM((G, B), jnp.float32),
                     pltpu.VMEM((G, B), jnp.float32)]

    @functools.partial(
        pl.kernel,
        mesh=mesh,
        out_type=out_t,
        scratch_types=[
            pltpu.VMEM((NCHUNK, G), jnp.int32),
            pltpu.VMEM((NCHUNK, G), jnp.int32),
            pltpu.VMEM((16, NPW), jnp.float32),
            pltpu.VMEM((NPW + LANES,), jnp.float32),
            pltpu.VMEM((NPW + LANES,), jnp.float32),
            pltpu.VMEM((NPW + LANES,), jnp.float32),
            pltpu.VMEM((NPW + LANES,), jnp.float32),
            pltpu.VMEM((2, G, B), jnp.float32),
            pltpu.VMEM((2, G, B), jnp.float32),
            *o_scratch,
            pltpu.SemaphoreType.DMA,
            pltpu.SemaphoreType.DMA,
            pltpu.SemaphoreType.DMA,
            pltpu.SemaphoreType.DMA,
            pltpu.SemaphoreType.DMA,
            pltpu.SemaphoreType.DMA,
        ],
    )
    def layer(h_hbm, ia_hbm, ib_hbm, wt_hbm, out_hbm,
              ia_v, ib_v, w_v, c0_v, ca_v, cb_v, cab_v,
              a_v, b_v, *o_and_sems):
        if final:
            o_v, sa0, sa1, sb0, sb1, so0, so1 = o_and_sems
            o_bufs = (o_v, o_v)
        else:
            o0, o1, sa0, sa1, sb0, sb1, so0, so1 = o_and_sems
            o_bufs = (o0, o1)
        sa = (sa0, sa1)
        sb = (sb0, sb1)
        so = (so0, so1)
        wid = lax.axis_index("s") * NC + lax.axis_index("c")
        base = wid * NPW
        pltpu.sync_copy(ia_hbm.at[pl.ds(wid * NCHUNK, NCHUNK)], ia_v)
        pltpu.sync_copy(ib_hbm.at[pl.ds(wid * NCHUNK, NCHUNK)], ib_v)
        pltpu.sync_copy(wt_hbm.at[:, pl.ds(base, NPW)], w_v)
        _compute_coeffs(w_v, c0_v, ca_v, cb_v, cab_v)

        if final:
            def zero_body(t, carry):
                sl = pl.ds(t * LANES, LANES)
                o_v[sl] = o_v[sl] * 0.0
                return carry
            lax.fori_loop(0, B // LANES, zero_body, 0)

        def in_copies(g, k):
            return (
                pltpu.make_async_copy(h_hbm.at[ia_v.at[g]], a_v.at[k], sa[k]),
                pltpu.make_async_copy(h_hbm.at[ib_v.at[g]], b_v.at[k], sb[k]),
            )

        def out_copy(g, k):
            return pltpu.make_async_copy(
                o_bufs[k], out_hbm.at[pl.ds(base + g * G, G)], so[k])

        def start_in(g, k):
            ca_, cb_ = in_copies(g, k)
            ca_.start()
            cb_.start()

        def wait_in(g, k):
            ca_, cb_ = in_copies(g, k)
            ca_.wait()
            cb_.wait()

        def compute(k, coefs, off):
            c0g, cag, cbg, cabg = coefs
            for gg in range(G):
                c0 = c0g[off + gg]
                ca_ = cag[off + gg]
                cb_ = cbg[off + gg]
                cab = cabg[off + gg]

                def inner(t, icarry):
                    sl = pl.ds(t * LANES, LANES)
                    a = a_v[k, gg, sl]
                    b = b_v[k, gg, sl]
                    r = (c0 + ca_ * a) + b * (cb_ + cab * a)
                    if final:
                        plsc.addupdate(o_bufs[k].at[sl], r)
                    else:
                        o_bufs[k][gg, sl] = r
                    return icarry

                lax.fori_loop(0, B // LANES, inner, 0, unroll=4)

        start_in(0, 0)

        def pair(gp, carry):
            g0 = gp * 2
            g1 = g0 + 1
            cs = pl.ds(gp * 2 * G, LANES)
            coefs = (c0_v[cs], ca_v[cs], cb_v[cs], cab_v[cs])
            start_in(g1, 1)
            wait_in(g0, 0)
            if not final:
                @pl.when(gp > 0)
                def _():
                    out_copy(g0 - 2, 0).wait()
            compute(0, coefs, 0)
            if not final:
                out_copy(g0, 0).start()

            @pl.when(gp < NCHUNK // 2 - 1)
            def _():
                start_in(g0 + 2, 0)
            wait_in(g1, 1)
            if not final:
                @pl.when(gp > 0)
                def _():
                    out_copy(g1 - 2, 1).wait()
            compute(1, coefs, G)
            if not final:
                out_copy(g1, 1).start()
            return carry

        lax.fori_loop(0, NCHUNK // 2, pair, 0)
        if final:
            pltpu.sync_copy(o_v, out_hbm.at[wid % 2, wid // 2])
        else:
            out_copy(NCHUNK - 2, 0).wait()
            out_copy(NCHUNK - 1, 1).wait()

    return layer


_layer0 = _make_layer(IN, False)
_layer_mid = _make_layer(N, False)
_layer_last = _make_layer(N, True)


def _combine_body(p_ref, o_ref):
    o_ref[...] = (p_ref[0] + p_ref[1]) * (1.0 / TAU)


def _combine(part):
    blk = 512
    return pl.pallas_call(
        _combine_body,
        grid=(B // blk,),
        in_specs=[pl.BlockSpec((2, C, blk), lambda i: (0, 0, i))],
        out_specs=pl.BlockSpec((C, blk), lambda i: (0, i)),
        out_shape=jax.ShapeDtypeStruct((C, B), jnp.float32),
    )(part)


def kernel(x, idx_a0, idx_b0, w0, idx_a1, idx_b1, w1,
           idx_a2, idx_b2, w2, idx_a3, idx_b3, w3):
    h = x.T
    r = lambda i: i.reshape(NW * NCHUNK, G)
    h = _layer0(h, r(idx_a0), r(idx_b0), w0.T)
    h = _layer_mid(h, r(idx_a1), r(idx_b1), w1.T)
    h = _layer_mid(h, r(idx_a2), r(idx_b2), w2.T)
    part = _layer_last(h, r(idx_a3), r(idx_b3), w3.T)
    return _combine(part).T


# speedup vs baseline: 2.9685x; 2.9685x over previous
import functools

import jax
import jax.numpy as jnp
from jax import lax
from jax.experimental import pallas as pl
from jax.experimental.pallas import tpu as pltpu
from jax.experimental.pallas import tpu_sc as plsc

B, IN, N, C, TAU = 4096, 1024, 8192, 16, 10.0
B2 = B // 2
NC, NS, LANES = 2, 16, 16
NW = NC * NS
NPW = N // NW
G = 8
NCHUNK = NPW // G


def _compute_coeffs(w_v, c0_v, ca_v, cb_v, cab_v):
    def group(q, carry):
        sl = pl.ds(q * LANES, LANES)
        rows = [w_v[g, sl] for g in range(16)]
        m = rows[0]
        for g in range(1, 16):
            m = jnp.maximum(m, rows[g])
        e = [jnp.exp(r - m) for r in rows]
        s = e[0]
        for g in range(1, 16):
            s = s + e[g]
        inv = 1.0 / s
        c0r = ((e[8] + e[9]) + (e[10] + e[11])) + ((e[12] + e[13]) + (e[14] + e[15]))
        car = ((e[2] + e[3]) + (e[6] + e[7])) - ((e[8] + e[9]) + (e[12] + e[13]))
        cbr = ((e[4] + e[5]) + (e[6] + e[7])) - ((e[8] + e[9]) + (e[10] + e[11]))
        cabr = (((e[1] + e[8]) + 2.0 * e[9]) + (e[11] + e[13])) - \
               (((e[2] + e[4]) + 2.0 * e[6]) + (e[7] + e[14]))
        c0_v[sl] = c0r * inv
        ca_v[sl] = car * inv
        cb_v[sl] = cbr * inv
        cab_v[sl] = cabr * inv
        return carry

    lax.fori_loop(0, NPW // LANES, group, 0)


def _unpack(wv):
    lo = plsc.bitcast(lax.shift_left(wv, 16), jnp.float32)
    hi = plsc.bitcast(wv, jnp.float32)
    return lo, hi


def _repack(rlo, rhi):
    bl = plsc.bitcast(rlo, jnp.int32)
    bh = plsc.bitcast(rhi, jnp.int32)
    rl = lax.shift_right_logical(bl + 0x8000, 16)
    rh = (bh + 0x8000) & (-65536)
    return rl | rh


def _make_layer(in_dim, final):
    mesh = plsc.VectorSubcoreMesh(core_axis_name="c", subcore_axis_name="s")
    if final:
        out_t = jax.ShapeDtypeStruct((2, C, B), jnp.float32)
        o_scratch = [pltpu.VMEM((B,), jnp.float32)]
    else:
        out_t = jax.ShapeDtypeStruct((N, B2), jnp.int32)
        o_scratch = [pltpu.VMEM((G, B2), jnp.int32),
                     pltpu.VMEM((G, B2), jnp.int32)]

    @functools.partial(
        pl.kernel,
        mesh=mesh,
        out_type=out_t,
        compiler_params=pltpu.CompilerParams(needs_layout_passes=False),
        scratch_types=[
            pltpu.VMEM((NCHUNK, G), jnp.int32),
            pltpu.VMEM((NCHUNK, G), jnp.int32),
            pltpu.VMEM((16, NPW), jnp.float32),
            pltpu.VMEM((NPW + LANES,), jnp.float32),
            pltpu.VMEM((NPW + LANES,), jnp.float32),
            pltpu.VMEM((NPW + LANES,), jnp.float32),
            pltpu.VMEM((NPW + LANES,), jnp.float32),
            pltpu.VMEM((2, G, B2), jnp.int32),
            pltpu.VMEM((2, G, B2), jnp.int32),
            *o_scratch,
            pltpu.SemaphoreType.DMA,
            pltpu.SemaphoreType.DMA,
            pltpu.SemaphoreType.DMA,
            pltpu.SemaphoreType.DMA,
            pltpu.SemaphoreType.DMA,
            pltpu.SemaphoreType.DMA,
        ],
    )
    def layer(h_hbm, ia_hbm, ib_hbm, wt_hbm, out_hbm,
              ia_v, ib_v, w_v, c0_v, ca_v, cb_v, cab_v,
              a_v, b_v, *o_and_sems):
        if final:
            o_v, sa0, sa1, sb0, sb1, so0, so1 = o_and_sems
            o_bufs = (o_v, o_v)
        else:
            o0, o1, sa0, sa1, sb0, sb1, so0, so1 = o_and_sems
            o_bufs = (o0, o1)
        sa = (sa0, sa1)
        sb = (sb0, sb1)
        so = (so0, so1)
        wid = lax.axis_index("s") * NC + lax.axis_index("c")
        base = wid * NPW
        pltpu.sync_copy(ia_hbm.at[pl.ds(wid * NCHUNK, NCHUNK)], ia_v)
        pltpu.sync_copy(ib_hbm.at[pl.ds(wid * NCHUNK, NCHUNK)], ib_v)

        def in_copies(g, k):
            return (
                pltpu.make_async_copy(h_hbm.at[ia_v.at[g]], a_v.at[k], sa[k]),
                pltpu.make_async_copy(h_hbm.at[ib_v.at[g]], b_v.at[k], sb[k]),
            )

        def out_copy(g, k):
            return pltpu.make_async_copy(
                o_bufs[k], out_hbm.at[pl.ds(base + g * G, G)], so[k])

        def start_in(g, k):
            ca_, cb_ = in_copies(g, k)
            ca_.start()
            cb_.start()

        def wait_in(g, k):
            ca_, cb_ = in_copies(g, k)
            ca_.wait()
            cb_.wait()

        start_in(0, 0)
        pltpu.sync_copy(wt_hbm.at[:, pl.ds(base, NPW)], w_v)
        _compute_coeffs(w_v, c0_v, ca_v, cb_v, cab_v)

        if final:
            def zero_body(t, carry):
                sl = pl.ds(t * LANES, LANES)
                o_v[sl] = o_v[sl] * 0.0
                return carry
            lax.fori_loop(0, B // LANES, zero_body, 0)

        def compute(k, coefs, off):
            c0g, cag, cbg, cabg = coefs
            for gg in range(G):
                c0 = c0g[off + gg]
                ca_ = cag[off + gg]
                cb_ = cbg[off + gg]
                cab = cabg[off + gg]

                @plsc.parallel_loop(0, B2 // LANES, unroll=8)
                def _(t):
                    sl = pl.ds(t * LANES, LANES)
                    alo, ahi = _unpack(a_v[k, gg, sl])
                    blo, bhi = _unpack(b_v[k, gg, sl])
                    rlo = (c0 + ca_ * alo) + blo * (cb_ + cab * alo)
                    rhi = (c0 + ca_ * ahi) + bhi * (cb_ + cab * ahi)
                    if final:
                        plsc.addupdate(o_v.at[pl.ds(t * 32, LANES)], rlo)
                        plsc.addupdate(o_v.at[pl.ds(t * 32 + LANES, LANES)], rhi)
                    else:
                        o_bufs[k][gg, sl] = _repack(rlo, rhi)

        def pair(gp, carry):
            g0 = gp * 2
            g1 = g0 + 1
            cs = pl.ds(gp * 2 * G, LANES)
            coefs = (c0_v[cs], ca_v[cs], cb_v[cs], cab_v[cs])
            start_in(g1, 1)
            wait_in(g0, 0)
            if not final:
                @pl.when(gp > 0)
                def _():
                    out_copy(g0 - 2, 0).wait()
            compute(0, coefs, 0)
            if not final:
                out_copy(g0, 0).start()

            @pl.when(gp < NCHUNK // 2 - 1)
            def _():
                start_in(g0 + 2, 0)
            wait_in(g1, 1)
            if not final:
                @pl.when(gp > 0)
                def _():
                    out_copy(g1 - 2, 1).wait()
            compute(1, coefs, G)
            if not final:
                out_copy(g1, 1).start()
            return carry

        lax.fori_loop(0, NCHUNK // 2, pair, 0)
        if final:
            pltpu.sync_copy(o_v, out_hbm.at[wid % 2, wid // 2])
        else:
            out_copy(NCHUNK - 2, 0).wait()
            out_copy(NCHUNK - 1, 1).wait()

    return layer


_layer0 = _make_layer(IN, False)
_layer_mid = _make_layer(N, False)
_layer_last = _make_layer(N, True)


def _combine_body(p_ref, o_ref):
    o_ref[...] = (p_ref[0] + p_ref[1]) * (1.0 / TAU)


def _combine(part):
    blk = 512
    return pl.pallas_call(
        _combine_body,
        grid=(B // blk,),
        in_specs=[pl.BlockSpec((2, C, blk), lambda i: (0, 0, i))],
        out_specs=pl.BlockSpec((C, blk), lambda i: (0, i)),
        out_shape=jax.ShapeDtypeStruct((C, B), jnp.float32),
    )(part)


def kernel(x, idx_a0, idx_b0, w0, idx_a1, idx_b1, w1,
           idx_a2, idx_b2, w2, idx_a3, idx_b3, w3):
    h = lax.bitcast_convert_type(
        x.T.astype(jnp.bfloat16).reshape(IN, B2, 2), jnp.int32)
    r = lambda i: i.reshape(NW * NCHUNK, G)
    h = _layer0(h, r(idx_a0), r(idx_b0), w0.T)
    h = _layer_mid(h, r(idx_a1), r(idx_b1), w1.T)
    h = _layer_mid(h, r(idx_a2), r(idx_b2), w2.T)
    part = _layer_last(h, r(idx_a3), r(idx_b3), w3.T)
    out = _combine(part)
    out = out.reshape(C, B // 32, 2, 16).transpose(0, 1, 3, 2).reshape(C, B)
    return out.T


# speedup vs baseline: 4.3423x; 1.4628x over previous
import functools

import jax
import jax.numpy as jnp
from jax import lax
from jax.experimental import pallas as pl
from jax.experimental.pallas import tpu as pltpu
from jax.experimental.pallas import tpu_sc as plsc

B, IN, N, C, TAU = 4096, 1024, 8192, 16, 10.0
B2 = B // 2
NC, NS, LANES = 2, 16, 16
NW = NC * NS
NPW = N // NW
G = 8
NCHUNK = NPW // G


def _compute_coeffs(w_v, c0_v, ca_v, cb_v, cab_v):
    def group(q, carry):
        sl = pl.ds(q * LANES, LANES)
        rows = [w_v[g, sl] for g in range(16)]
        m = rows[0]
        for g in range(1, 16):
            m = jnp.maximum(m, rows[g])
        e = [jnp.exp(r - m) for r in rows]
        s = e[0]
        for g in range(1, 16):
            s = s + e[g]
        inv = 1.0 / s
        c0r = ((e[8] + e[9]) + (e[10] + e[11])) + ((e[12] + e[13]) + (e[14] + e[15]))
        car = ((e[2] + e[3]) + (e[6] + e[7])) - ((e[8] + e[9]) + (e[12] + e[13]))
        cbr = ((e[4] + e[5]) + (e[6] + e[7])) - ((e[8] + e[9]) + (e[10] + e[11]))
        cabr = (((e[1] + e[8]) + 2.0 * e[9]) + (e[11] + e[13])) - \
               (((e[2] + e[4]) + 2.0 * e[6]) + (e[7] + e[14]))
        c0_v[sl] = c0r * inv
        ca_v[sl] = car * inv
        cb_v[sl] = cbr * inv
        cab_v[sl] = cabr * inv
        return carry

    lax.fori_loop(0, NPW // LANES, group, 0)


def _unpack(wv):
    lo = plsc.bitcast(lax.shift_left(wv, 16), jnp.float32)
    hi = plsc.bitcast(wv, jnp.float32)
    return lo, hi


def _repack(rlo, rhi):
    bl = plsc.bitcast(rlo, jnp.int32)
    bh = plsc.bitcast(rhi, jnp.int32)
    rl = lax.shift_right_logical(bl + 0x8000, 16)
    rh = (bh + 0x8000) & (-65536)
    return rl | rh


def _make_layer(in_dim, final):
    mesh = plsc.VectorSubcoreMesh(core_axis_name="c", subcore_axis_name="s")
    if final:
        out_t = jax.ShapeDtypeStruct((2, C, B), jnp.float32)
        o_scratch = [pltpu.VMEM((B,), jnp.float32)]
    else:
        out_t = jax.ShapeDtypeStruct((N, B2), jnp.int32)
        o_scratch = [pltpu.VMEM((G, B2), jnp.int32),
                     pltpu.VMEM((G, B2), jnp.int32)]

    @functools.partial(
        pl.kernel,
        mesh=mesh,
        out_type=out_t,
        compiler_params=pltpu.CompilerParams(needs_layout_passes=False),
        scratch_types=[
            pltpu.VMEM((NCHUNK, G), jnp.int32),
            pltpu.VMEM((NCHUNK, G), jnp.int32),
            pltpu.VMEM((16, NPW), jnp.float32),
            pltpu.VMEM((NPW + LANES,), jnp.float32),
            pltpu.VMEM((NPW + LANES,), jnp.float32),
            pltpu.VMEM((NPW + LANES,), jnp.float32),
            pltpu.VMEM((NPW + LANES,), jnp.float32),
            pltpu.VMEM((2, G, B2), jnp.int32),
            pltpu.VMEM((2, G, B2), jnp.int32),
            *o_scratch,
            pltpu.SemaphoreType.DMA,
            pltpu.SemaphoreType.DMA,
            pltpu.SemaphoreType.DMA,
            pltpu.SemaphoreType.DMA,
            pltpu.SemaphoreType.DMA,
            pltpu.SemaphoreType.DMA,
        ],
    )
    def layer(h_hbm, ia_hbm, ib_hbm, wt_hbm, out_hbm,
              ia_v, ib_v, w_v, c0_v, ca_v, cb_v, cab_v,
              a_v, b_v, *o_and_sems):
        if final:
            o_v, sa0, sa1, sb0, sb1, so0, so1 = o_and_sems
            o_bufs = (o_v, o_v)
        else:
            o0, o1, sa0, sa1, sb0, sb1, so0, so1 = o_and_sems
            o_bufs = (o0, o1)
        sa = (sa0, sa1)
        sb = (sb0, sb1)
        so = (so0, so1)
        wid = lax.axis_index("s") * NC + lax.axis_index("c")
        base = wid * NPW
        pltpu.sync_copy(ia_hbm.at[pl.ds(wid * NCHUNK, NCHUNK)], ia_v)
        pltpu.sync_copy(ib_hbm.at[pl.ds(wid * NCHUNK, NCHUNK)], ib_v)

        def in_copies(g, k):
            return (
                pltpu.make_async_copy(h_hbm.at[ia_v.at[g]], a_v.at[k], sa[k]),
                pltpu.make_async_copy(h_hbm.at[ib_v.at[g]], b_v.at[k], sb[k]),
            )

        def out_copy(g, k):
            return pltpu.make_async_copy(
                o_bufs[k], out_hbm.at[pl.ds(base + g * G, G)], so[k])

        def start_in(g, k):
            ca_, cb_ = in_copies(g, k)
            ca_.start()
            cb_.start()

        def wait_in(g, k):
            ca_, cb_ = in_copies(g, k)
            ca_.wait()
            cb_.wait()

        start_in(0, 0)
        pltpu.sync_copy(wt_hbm.at[:, pl.ds(base, NPW)], w_v)
        _compute_coeffs(w_v, c0_v, ca_v, cb_v, cab_v)

        if final:
            def zero_body(t, carry):
                sl = pl.ds(t * LANES, LANES)
                o_v[sl] = o_v[sl] * 0.0
                return carry
            lax.fori_loop(0, B // LANES, zero_body, 0)

        def splat_bf(s):
            ci = lax.bitcast_convert_type(s, jnp.int32)
            t = lax.shift_right_logical(ci + 0x8000, 16)
            p = jnp.bitwise_or(t, lax.shift_left(t, 16))
            return plsc.bitcast(jnp.broadcast_to(p, (LANES,)), jnp.bfloat16)

        def compute(k, coefs, off):
            c0g, cag, cbg, cabg = coefs
            for gg in range(G):
                c0 = splat_bf(c0g[off + gg])
                ca_ = splat_bf(cag[off + gg])
                cb_ = splat_bf(cbg[off + gg])
                cab = splat_bf(cabg[off + gg])

                @plsc.parallel_loop(0, B2 // LANES, unroll=8)
                def _(t):
                    sl = pl.ds(t * LANES, LANES)
                    a = plsc.bitcast(a_v[k, gg, sl], jnp.bfloat16)
                    b = plsc.bitcast(b_v[k, gg, sl], jnp.bfloat16)
                    r = (c0 + ca_ * a) + b * (cb_ + cab * a)
                    if final:
                        ri = plsc.bitcast(r, jnp.int32)
                        lo = plsc.bitcast(lax.shift_left(ri, 16), jnp.float32)
                        hi = plsc.bitcast(ri & (-65536), jnp.float32)
                        plsc.addupdate(o_v.at[pl.ds(t * 32, LANES)], lo)
                        plsc.addupdate(o_v.at[pl.ds(t * 32 + LANES, LANES)], hi)
                    else:
                        o_bufs[k][gg, sl] = plsc.bitcast(r, jnp.int32)

        def pair(gp, carry):
            g0 = gp * 2
            g1 = g0 + 1
            cs = pl.ds(gp * 2 * G, LANES)
            coefs = (c0_v[cs], ca_v[cs], cb_v[cs], cab_v[cs])
            start_in(g1, 1)
            wait_in(g0, 0)
            if not final:
                @pl.when(gp > 0)
                def _():
                    out_copy(g0 - 2, 0).wait()
            compute(0, coefs, 0)
            if not final:
                out_copy(g0, 0).start()

            @pl.when(gp < NCHUNK // 2 - 1)
            def _():
                start_in(g0 + 2, 0)
            wait_in(g1, 1)
            if not final:
                @pl.when(gp > 0)
                def _():
                    out_copy(g1 - 2, 1).wait()
            compute(1, coefs, G)
            if not final:
                out_copy(g1, 1).start()
            return carry

        lax.fori_loop(0, NCHUNK // 2, pair, 0)
        if final:
            pltpu.sync_copy(o_v, out_hbm.at[wid % 2, wid // 2])
        else:
            out_copy(NCHUNK - 2, 0).wait()
            out_copy(NCHUNK - 1, 1).wait()

    return layer


_layer0 = _make_layer(IN, False)
_layer_mid = _make_layer(N, False)
_layer_last = _make_layer(N, True)


def _combine_body(p_ref, o_ref):
    o_ref[...] = (p_ref[0] + p_ref[1]) * (1.0 / TAU)


def _combine(part):
    blk = 512
    return pl.pallas_call(
        _combine_body,
        grid=(B // blk,),
        in_specs=[pl.BlockSpec((2, C, blk), lambda i: (0, 0, i))],
        out_specs=pl.BlockSpec((C, blk), lambda i: (0, i)),
        out_shape=jax.ShapeDtypeStruct((C, B), jnp.float32),
    )(part)


def kernel(x, idx_a0, idx_b0, w0, idx_a1, idx_b1, w1,
           idx_a2, idx_b2, w2, idx_a3, idx_b3, w3):
    h = lax.bitcast_convert_type(
        x.T.astype(jnp.bfloat16).reshape(IN, B2, 2), jnp.int32)
    r = lambda i: i.reshape(NW * NCHUNK, G)
    h = _layer0(h, r(idx_a0), r(idx_b0), w0.T)
    h = _layer_mid(h, r(idx_a1), r(idx_b1), w1.T)
    h = _layer_mid(h, r(idx_a2), r(idx_b2), w2.T)
    part = _layer_last(h, r(idx_a3), r(idx_b3), w3.T)
    out = _combine(part)
    out = out.reshape(C, B // 32, 2, 16).transpose(0, 1, 3, 2).reshape(C, B)
    return out.T


# speedup vs baseline: 4.8815x; 1.1242x over previous
import functools

import jax
import jax.numpy as jnp
from jax import lax
from jax.experimental import pallas as pl
from jax.experimental.pallas import tpu as pltpu
from jax.experimental.pallas import tpu_sc as plsc

B, IN, N, C, TAU = 4096, 1024, 8192, 16, 10.0
B2 = B // 2
NC, NS, LANES = 2, 16, 16
NW = NC * NS
NPW = N // NW
G = 8
NCHUNK = NPW // G


def _compute_coeffs(w_v, c0_v, ca_v, cb_v, cab_v):
    def group(q, carry):
        sl = pl.ds(q * LANES, LANES)
        rows = [w_v[g, sl] for g in range(16)]
        m = rows[0]
        for g in range(1, 16):
            m = jnp.maximum(m, rows[g])
        e = [jnp.exp(r - m) for r in rows]
        s = e[0]
        for g in range(1, 16):
            s = s + e[g]
        inv = 1.0 / s
        c0r = ((e[8] + e[9]) + (e[10] + e[11])) + ((e[12] + e[13]) + (e[14] + e[15]))
        car = ((e[2] + e[3]) + (e[6] + e[7])) - ((e[8] + e[9]) + (e[12] + e[13]))
        cbr = ((e[4] + e[5]) + (e[6] + e[7])) - ((e[8] + e[9]) + (e[10] + e[11]))
        cabr = (((e[1] + e[8]) + 2.0 * e[9]) + (e[11] + e[13])) - \
               (((e[2] + e[4]) + 2.0 * e[6]) + (e[7] + e[14]))
        c0_v[sl] = c0r * inv
        ca_v[sl] = car * inv
        cb_v[sl] = cbr * inv
        cab_v[sl] = cabr * inv
        return carry

    lax.fori_loop(0, NPW // LANES, group, 0)


def _unpack(wv):
    lo = plsc.bitcast(lax.shift_left(wv, 16), jnp.float32)
    hi = plsc.bitcast(wv, jnp.float32)
    return lo, hi


def _repack(rlo, rhi):
    bl = plsc.bitcast(rlo, jnp.int32)
    bh = plsc.bitcast(rhi, jnp.int32)
    rl = lax.shift_right_logical(bl + 0x8000, 16)
    rh = (bh + 0x8000) & (-65536)
    return rl | rh


def _make_layer(in_dim, final):
    mesh = plsc.VectorSubcoreMesh(core_axis_name="c", subcore_axis_name="s")
    if final:
        out_t = jax.ShapeDtypeStruct((2, C, B), jnp.float32)
        o_scratch = [pltpu.VMEM((B,), jnp.float32)]
    else:
        out_t = jax.ShapeDtypeStruct((N, B2), jnp.int32)
        o_scratch = [pltpu.VMEM((G, B2), jnp.int32),
                     pltpu.VMEM((G, B2), jnp.int32)]

    @functools.partial(
        pl.kernel,
        mesh=mesh,
        out_type=out_t,
        compiler_params=pltpu.CompilerParams(needs_layout_passes=False),
        scratch_types=[
            pltpu.VMEM((NCHUNK, G), jnp.int32),
            pltpu.VMEM((NCHUNK, G), jnp.int32),
            pltpu.VMEM((16, NPW), jnp.float32),
            pltpu.VMEM((NPW + LANES,), jnp.float32),
            pltpu.VMEM((NPW + LANES,), jnp.float32),
            pltpu.VMEM((NPW + LANES,), jnp.float32),
            pltpu.VMEM((NPW + LANES,), jnp.float32),
            pltpu.VMEM((2, G, B2), jnp.int32),
            pltpu.VMEM((2, G, B2), jnp.int32),
            *o_scratch,
            pltpu.SemaphoreType.DMA,
            pltpu.SemaphoreType.DMA,
            pltpu.SemaphoreType.DMA,
            pltpu.SemaphoreType.DMA,
            pltpu.SemaphoreType.DMA,
            pltpu.SemaphoreType.DMA,
        ],
    )
    def layer(h_hbm, ia_hbm, ib_hbm, wt_hbm, out_hbm,
              ia_v, ib_v, w_v, c0_v, ca_v, cb_v, cab_v,
              a_v, b_v, *o_and_sems):
        if final:
            o_v, sa0, sa1, sb0, sb1, so0, so1 = o_and_sems
            o_bufs = (o_v, o_v)
        else:
            o0, o1, sa0, sa1, sb0, sb1, so0, so1 = o_and_sems
            o_bufs = (o0, o1)
        sa = (sa0, sa1)
        sb = (sb0, sb1)
        so = (so0, so1)
        wid = lax.axis_index("s") * NC + lax.axis_index("c")
        base = wid * NPW
        pltpu.sync_copy(ia_hbm.at[pl.ds(wid * NCHUNK, NCHUNK)], ia_v)
        pltpu.sync_copy(ib_hbm.at[pl.ds(wid * NCHUNK, NCHUNK)], ib_v)

        def in_copies(g, k):
            return (
                pltpu.make_async_copy(h_hbm.at[ia_v.at[g]], a_v.at[k], sa[k]),
                pltpu.make_async_copy(h_hbm.at[ib_v.at[g]], b_v.at[k], sb[k]),
            )

        def out_copy(g, k):
            return pltpu.make_async_copy(
                o_bufs[k], out_hbm.at[pl.ds(base + g * G, G)], so[k])

        def start_in(g, k):
            ca_, cb_ = in_copies(g, k)
            ca_.start()
            cb_.start()

        def wait_in(g, k):
            ca_, cb_ = in_copies(g, k)
            ca_.wait()
            cb_.wait()

        start_in(0, 0)
        pltpu.sync_copy(wt_hbm.at[:, pl.ds(base, NPW)], w_v)
        _compute_coeffs(w_v, c0_v, ca_v, cb_v, cab_v)

        if final:
            def zero_body(t, carry):
                sl = pl.ds(t * LANES, LANES)
                o_v[sl] = o_v[sl] * 0.0
                return carry
            lax.fori_loop(0, B // LANES, zero_body, 0)

        def splat_bf(s):
            ci = lax.bitcast_convert_type(s, jnp.int32)
            t = lax.shift_right_logical(ci + 0x8000, 16)
            p = jnp.bitwise_or(t, lax.shift_left(t, 16))
            return plsc.bitcast(jnp.broadcast_to(p, (LANES,)), jnp.bfloat16)

        def compute(k, coefs, off):
            c0g, cag, cbg, cabg = coefs
            for gg in range(G):
                c0 = splat_bf(c0g[off + gg])
                ca_ = splat_bf(cag[off + gg])
                cb_ = splat_bf(cbg[off + gg])
                cab = splat_bf(cabg[off + gg])

                @plsc.parallel_loop(0, B2 // LANES, unroll=8)
                def _(t):
                    sl = pl.ds(t * LANES, LANES)
                    a = plsc.bitcast(a_v[k, gg, sl], jnp.bfloat16)
                    b = plsc.bitcast(b_v[k, gg, sl], jnp.bfloat16)
                    r = (c0 + ca_ * a) + b * (cb_ + cab * a)
                    if final:
                        ri = plsc.bitcast(r, jnp.int32)
                        lo = plsc.bitcast(lax.shift_left(ri, 16), jnp.float32)
                        hi = plsc.bitcast(ri & (-65536), jnp.float32)
                        plsc.addupdate(o_v.at[pl.ds(t * 32, LANES)], lo)
                        plsc.addupdate(o_v.at[pl.ds(t * 32 + LANES, LANES)], hi)
                    else:
                        o_bufs[k][gg, sl] = plsc.bitcast(r, jnp.int32)

        def pair(gp, carry):
            g0 = gp * 2
            g1 = g0 + 1
            cs = pl.ds(gp * 2 * G, LANES)
            coefs = (c0_v[cs], ca_v[cs], cb_v[cs], cab_v[cs])
            start_in(g1, 1)
            wait_in(g0, 0)
            if not final:
                @pl.when(gp > 0)
                def _():
                    out_copy(g0 - 2, 0).wait()
            compute(0, coefs, 0)
            if not final:
                out_copy(g0, 0).start()

            @pl.when(gp < NCHUNK // 2 - 1)
            def _():
                start_in(g0 + 2, 0)
            wait_in(g1, 1)
            if not final:
                @pl.when(gp > 0)
                def _():
                    out_copy(g1 - 2, 1).wait()
            compute(1, coefs, G)
            if not final:
                out_copy(g1, 1).start()
            return carry

        lax.fori_loop(0, NCHUNK // 2, pair, 0)
        if final:
            pltpu.sync_copy(o_v, out_hbm.at[wid % 2, wid // 2])
        else:
            out_copy(NCHUNK - 2, 0).wait()
            out_copy(NCHUNK - 1, 1).wait()

    return layer


_layer0 = _make_layer(IN, False)
_layer_mid = _make_layer(N, False)
_layer_last = _make_layer(N, True)

RPW = IN // NW
RCH = 16


def _make_pack():
    mesh = plsc.VectorSubcoreMesh(core_axis_name="c", subcore_axis_name="s")

    @functools.partial(
        pl.kernel,
        mesh=mesh,
        out_type=jax.ShapeDtypeStruct((IN, B2), jnp.int32),
        compiler_params=pltpu.CompilerParams(needs_layout_passes=False),
        scratch_types=[
            pltpu.VMEM((RCH, B), jnp.float32),
            pltpu.VMEM((RCH, B2), jnp.int32),
        ],
    )
    def packk(x_hbm, out_hbm, in_v, out_v):
        wid = lax.axis_index("s") * NC + lax.axis_index("c")
        base = wid * RPW

        def chunk(cidx, carry):
            row0 = base + cidx * RCH
            pltpu.sync_copy(x_hbm.at[pl.ds(row0, RCH)], in_v)
            for rr in range(RCH):
                @plsc.parallel_loop(0, B2 // LANES, unroll=8)
                def _(t):
                    v0 = plsc.bitcast(in_v[rr, pl.ds(t * 32, LANES)], jnp.int32)
                    v1 = plsc.bitcast(in_v[rr, pl.ds(t * 32 + LANES, LANES)],
                                      jnp.int32)
                    lo = lax.shift_right_logical(v0 + 0x8000, 16)
                    hi = (v1 + 0x8000) & (-65536)
                    out_v[rr, pl.ds(t * LANES, LANES)] = lo | hi
            pltpu.sync_copy(out_v, out_hbm.at[pl.ds(row0, RCH)])
            return carry

        lax.fori_loop(0, RPW // RCH, chunk, 0)

    return packk


_pack = _make_pack()


def _combine_body(p_ref, o_ref):
    o_ref[...] = (p_ref[0] + p_ref[1]) * (1.0 / TAU)


def _combine(part):
    blk = 512
    return pl.pallas_call(
        _combine_body,
        grid=(B // blk,),
        in_specs=[pl.BlockSpec((2, C, blk), lambda i: (0, 0, i))],
        out_specs=pl.BlockSpec((C, blk), lambda i: (0, i)),
        out_shape=jax.ShapeDtypeStruct((C, B), jnp.float32),
    )(part)


def kernel(x, idx_a0, idx_b0, w0, idx_a1, idx_b1, w1,
           idx_a2, idx_b2, w2, idx_a3, idx_b3, w3):
    h = _pack(x.T)
    r = lambda i: i.reshape(NW * NCHUNK, G)
    h = _layer0(h, r(idx_a0), r(idx_b0), w0.T)
    h = _layer_mid(h, r(idx_a1), r(idx_b1), w1.T)
    h = _layer_mid(h, r(idx_a2), r(idx_b2), w2.T)
    part = _layer_last(h, r(idx_a3), r(idx_b3), w3.T)
    return _combine(part).T


# speedup vs baseline: 5.3188x; 1.0896x over previous
import functools

import jax
import jax.numpy as jnp
from jax import lax
from jax.experimental import pallas as pl
from jax.experimental.pallas import tpu as pltpu
from jax.experimental.pallas import tpu_sc as plsc

B, IN, N, C, TAU = 4096, 1024, 8192, 16, 10.0
B2 = B // 2
NC, NS, LANES = 2, 16, 16
NW = NC * NS
NPW = N // NW
G = 8
NCHUNK = NPW // G


def _compute_coeffs(w_v, c0_v, ca_v, cb_v, cab_v):
    def group(q, carry):
        sl = pl.ds(q * LANES, LANES)
        rows = [w_v[g, sl] for g in range(16)]
        m = rows[0]
        for g in range(1, 16):
            m = jnp.maximum(m, rows[g])
        e = [jnp.exp(r - m) for r in rows]
        s = e[0]
        for g in range(1, 16):
            s = s + e[g]
        inv = 1.0 / s
        c0r = ((e[8] + e[9]) + (e[10] + e[11])) + ((e[12] + e[13]) + (e[14] + e[15]))
        car = ((e[2] + e[3]) + (e[6] + e[7])) - ((e[8] + e[9]) + (e[12] + e[13]))
        cbr = ((e[4] + e[5]) + (e[6] + e[7])) - ((e[8] + e[9]) + (e[10] + e[11]))
        cabr = (((e[1] + e[8]) + 2.0 * e[9]) + (e[11] + e[13])) - \
               (((e[2] + e[4]) + 2.0 * e[6]) + (e[7] + e[14]))
        c0_v[sl] = c0r * inv
        ca_v[sl] = car * inv
        cb_v[sl] = cbr * inv
        cab_v[sl] = cabr * inv
        return carry

    lax.fori_loop(0, NPW // LANES, group, 0)


def _unpack(wv):
    lo = plsc.bitcast(lax.shift_left(wv, 16), jnp.float32)
    hi = plsc.bitcast(wv, jnp.float32)
    return lo, hi


def _repack(rlo, rhi):
    bl = plsc.bitcast(rlo, jnp.int32)
    bh = plsc.bitcast(rhi, jnp.int32)
    rl = lax.shift_right_logical(bl + 0x8000, 16)
    rh = (bh + 0x8000) & (-65536)
    return rl | rh


def _make_layer(in_dim, final):
    mesh = plsc.VectorSubcoreMesh(core_axis_name="c", subcore_axis_name="s")
    if final:
        out_t = jax.ShapeDtypeStruct((2, C, B), jnp.float32)
        o_scratch = [pltpu.VMEM((B,), jnp.float32)]
    else:
        out_t = jax.ShapeDtypeStruct((N, B2), jnp.int32)
        o_scratch = [pltpu.VMEM((G, B2), jnp.int32),
                     pltpu.VMEM((G, B2), jnp.int32)]

    @functools.partial(
        pl.kernel,
        mesh=mesh,
        out_type=out_t,
        compiler_params=pltpu.CompilerParams(needs_layout_passes=False),
        scratch_types=[
            pltpu.VMEM((NCHUNK, 2 * G), jnp.int32),
            pltpu.VMEM((16, NPW), jnp.float32),
            pltpu.VMEM((NPW + LANES,), jnp.float32),
            pltpu.VMEM((NPW + LANES,), jnp.float32),
            pltpu.VMEM((NPW + LANES,), jnp.float32),
            pltpu.VMEM((NPW + LANES,), jnp.float32),
            pltpu.VMEM((2, 2 * G, B2), jnp.int32),
            *o_scratch,
            pltpu.SemaphoreType.DMA,
            pltpu.SemaphoreType.DMA,
            pltpu.SemaphoreType.DMA,
            pltpu.SemaphoreType.DMA,
        ],
    )
    def layer(h_hbm, iab_hbm, wt_hbm, out_hbm,
              iab_v, w_v, c0_v, ca_v, cb_v, cab_v,
              ab_v, *o_and_sems):
        if final:
            o_v, sa0, sa1, so0, so1 = o_and_sems
            o_bufs = (o_v, o_v)
        else:
            o0, o1, sa0, sa1, so0, so1 = o_and_sems
            o_bufs = (o0, o1)
        sa = (sa0, sa1)
        so = (so0, so1)
        wid = lax.axis_index("s") * NC + lax.axis_index("c")
        base = wid * NPW
        pltpu.sync_copy(iab_hbm.at[pl.ds(wid * NCHUNK, NCHUNK)], iab_v)

        def in_copy(g, k):
            return pltpu.make_async_copy(h_hbm.at[iab_v.at[g]], ab_v.at[k], sa[k])

        def out_copy(g, k):
            return pltpu.make_async_copy(
                o_bufs[k], out_hbm.at[pl.ds(base + g * G, G)], so[k])

        def start_in(g, k):
            in_copy(g, k).start()

        def wait_in(g, k):
            in_copy(g, k).wait()

        start_in(0, 0)
        pltpu.sync_copy(wt_hbm.at[:, pl.ds(base, NPW)], w_v)
        _compute_coeffs(w_v, c0_v, ca_v, cb_v, cab_v)

        if final:
            def zero_body(t, carry):
                sl = pl.ds(t * LANES, LANES)
                o_v[sl] = o_v[sl] * 0.0
                return carry
            lax.fori_loop(0, B // LANES, zero_body, 0)

        def splat_bf(s):
            ci = lax.bitcast_convert_type(s, jnp.int32)
            t = lax.shift_right_logical(ci + 0x8000, 16)
            p = jnp.bitwise_or(t, lax.shift_left(t, 16))
            return plsc.bitcast(jnp.broadcast_to(p, (LANES,)), jnp.bfloat16)

        def body(k, gg, t, c0, ca_, cb_, cab):
            sl = pl.ds(t * LANES, LANES)
            a = plsc.bitcast(ab_v[k, gg, sl], jnp.bfloat16)
            b = plsc.bitcast(ab_v[k, G + gg, sl], jnp.bfloat16)
            return (c0 + ca_ * a) + b * (cb_ + cab * a)

        def compute(k, coefs, off):
            c0g, cag, cbg, cabg = coefs
            if final:
                cs = [(splat_bf(c0g[off + gg]), splat_bf(cag[off + gg]),
                       splat_bf(cbg[off + gg]), splat_bf(cabg[off + gg]))
                      for gg in range(G)]

                @plsc.parallel_loop(0, B2 // LANES, unroll=2)
                def _(t):
                    s = body(k, 0, t, *cs[0])
                    for gg in range(1, G):
                        s = s + body(k, gg, t, *cs[gg])
                    ri = plsc.bitcast(s, jnp.int32)
                    lo = plsc.bitcast(lax.shift_left(ri, 16), jnp.float32)
                    hi = plsc.bitcast(ri & (-65536), jnp.float32)
                    plsc.addupdate(o_v.at[pl.ds(t * 32, LANES)], lo)
                    plsc.addupdate(o_v.at[pl.ds(t * 32 + LANES, LANES)], hi)
            else:
                for gg in range(G):
                    c0 = splat_bf(c0g[off + gg])
                    ca_ = splat_bf(cag[off + gg])
                    cb_ = splat_bf(cbg[off + gg])
                    cab = splat_bf(cabg[off + gg])

                    @plsc.parallel_loop(0, B2 // LANES, unroll=8)
                    def _(t):
                        r = body(k, gg, t, c0, ca_, cb_, cab)
                        o_bufs[k][gg, pl.ds(t * LANES, LANES)] = \
                            plsc.bitcast(r, jnp.int32)

        def pair(gp, carry):
            g0 = gp * 2
            g1 = g0 + 1
            cs = pl.ds(gp * 2 * G, LANES)
            coefs = (c0_v[cs], ca_v[cs], cb_v[cs], cab_v[cs])
            start_in(g1, 1)
            wait_in(g0, 0)
            if not final:
                @pl.when(gp > 0)
                def _():
                    out_copy(g0 - 2, 0).wait()
            compute(0, coefs, 0)
            if not final:
                out_copy(g0, 0).start()

            @pl.when(gp < NCHUNK // 2 - 1)
            def _():
                start_in(g0 + 2, 0)
            wait_in(g1, 1)
            if not final:
                @pl.when(gp > 0)
                def _():
                    out_copy(g1 - 2, 1).wait()
            compute(1, coefs, G)
            if not final:
                out_copy(g1, 1).start()
            return carry

        lax.fori_loop(0, NCHUNK // 2, pair, 0)
        if final:
            pltpu.sync_copy(o_v, out_hbm.at[wid % 2, wid // 2])
        else:
            out_copy(NCHUNK - 2, 0).wait()
            out_copy(NCHUNK - 1, 1).wait()

    return layer


_layer0 = _make_layer(IN, False)
_layer_mid = _make_layer(N, False)
_layer_last = _make_layer(N, True)

RPW = IN // NW
RCH = 16


def _make_pack():
    mesh = plsc.VectorSubcoreMesh(core_axis_name="c", subcore_axis_name="s")

    @functools.partial(
        pl.kernel,
        mesh=mesh,
        out_type=jax.ShapeDtypeStruct((IN, B2), jnp.int32),
        compiler_params=pltpu.CompilerParams(needs_layout_passes=False),
        scratch_types=[
            pltpu.VMEM((RCH, B), jnp.float32),
            pltpu.VMEM((RCH, B2), jnp.int32),
        ],
    )
    def packk(x_hbm, out_hbm, in_v, out_v):
        wid = lax.axis_index("s") * NC + lax.axis_index("c")
        base = wid * RPW

        def chunk(cidx, carry):
            row0 = base + cidx * RCH
            pltpu.sync_copy(x_hbm.at[pl.ds(row0, RCH)], in_v)
            for rr in range(RCH):
                @plsc.parallel_loop(0, B2 // LANES, unroll=8)
                def _(t):
                    v0 = plsc.bitcast(in_v[rr, pl.ds(t * 32, LANES)], jnp.int32)
                    v1 = plsc.bitcast(in_v[rr, pl.ds(t * 32 + LANES, LANES)],
                                      jnp.int32)
                    lo = lax.shift_right_logical(v0 + 0x8000, 16)
                    hi = (v1 + 0x8000) & (-65536)
                    out_v[rr, pl.ds(t * LANES, LANES)] = lo | hi
            pltpu.sync_copy(out_v, out_hbm.at[pl.ds(row0, RCH)])
            return carry

        lax.fori_loop(0, RPW // RCH, chunk, 0)

    return packk


_pack = _make_pack()


def _combine_body(p_ref, o_ref):
    o_ref[...] = (p_ref[0] + p_ref[1]) * (1.0 / TAU)


def _combine(part):
    blk = 512
    return pl.pallas_call(
        _combine_body,
        grid=(B // blk,),
        in_specs=[pl.BlockSpec((2, C, blk), lambda i: (0, 0, i))],
        out_specs=pl.BlockSpec((C, blk), lambda i: (0, i)),
        out_shape=jax.ShapeDtypeStruct((C, B), jnp.float32),
    )(part)


def kernel(x, idx_a0, idx_b0, w0, idx_a1, idx_b1, w1,
           idx_a2, idx_b2, w2, idx_a3, idx_b3, w3):
    h = _pack(x.T)
    r = lambda ia, ib: jnp.concatenate(
        [ia.reshape(NW * NCHUNK, G), ib.reshape(NW * NCHUNK, G)], axis=1)
    h = _layer0(h, r(idx_a0, idx_b0), w0.T)
    h = _layer_mid(h, r(idx_a1, idx_b1), w1.T)
    h = _layer_mid(h, r(idx_a2, idx_b2), w2.T)
    part = _layer_last(h, r(idx_a3, idx_b3), w3.T)
    return _combine(part).T
